# baseline (speedup 1.0000x reference)
import jax
import jax.numpy as jnp
from jax import lax
from jax.experimental import pallas as pl
from jax.experimental.pallas import tpu as pltpu
from jax.experimental.pallas import tpu_sc as plsc

_NSRC = 32
_NY = 1024
_NX = 2048
_L = 16


def _sc_body(yref, y_hbm, x_hbm, upd_hbm, yv, xv, updv, tile, sem):
    cid = lax.axis_index("c")
    sid = lax.axis_index("s")
    wid = sid * 2 + cid

    pltpu.sync_copy(y_hbm, yv)
    pltpu.sync_copy(x_hbm, xv)
    pltpu.sync_copy(upd_hbm, updv)

    lanes = lax.iota(jnp.int32, _L)
    zero = jnp.zeros((_L,), jnp.int32)
    yi = jnp.int32(0)
    xi = jnp.int32(0)
    for c in range(_NSRC // _L):
        m = (lanes + c * _L) == wid
        yi = yi + jnp.sum(jnp.where(m, yv[pl.ds(c * _L, _L)], zero))
        xi = xi + jnp.sum(jnp.where(m, xv[pl.ds(c * _L, _L)], zero))
    row0 = wid * _NY + (yi >> 3) * 8
    col0 = (xi >> 7) * 128
    ry = yi & 7
    c0 = (xi & 127) & ~15
    lane = xi & 15

    pltpu.async_copy(yref.at[pl.ds(row0, 8), pl.ds(col0, 128)], tile, sem).wait()
    sel = lax.iota(jnp.int32, _L) == lane
    delta = jnp.where(sel, updv[...], jnp.float32(0.0))
    tile[ry, pl.ds(c0, _L)] = tile[ry, pl.ds(c0, _L)] + delta
    pltpu.async_copy(tile, yref.at[pl.ds(row0, 8), pl.ds(col0, 128)], sem).wait()


_scatter_add = pl.kernel(
    _sc_body,
    out_type=(),
    mesh=plsc.VectorSubcoreMesh(core_axis_name="c", subcore_axis_name="s"),
    scratch_types=[
        pltpu.VMEM((_NSRC,), jnp.int32),
        pltpu.VMEM((_NSRC,), jnp.int32),
        pltpu.VMEM((_L,), jnp.float32),
        pltpu.VMEM((8, 128), jnp.float32),
        pltpu.SemaphoreType.DMA,
    ],
    compiler_params=pltpu.CompilerParams(
        use_tc_tiling_on_sc=True, needs_layout_passes=False
    ),
)


def _copy_body(src, dst):
    dst[...] = src[...]


_ROWS_PER_BLOCK = 512
_tc_copy = pl.pallas_call(
    _copy_body,
    grid=(_NSRC * _NY // _ROWS_PER_BLOCK,),
    in_specs=[pl.BlockSpec((_ROWS_PER_BLOCK, _NX), lambda i: (i, 0))],
    out_specs=pl.BlockSpec((_ROWS_PER_BLOCK, _NX), lambda i: (i, 0)),
    out_shape=jax.ShapeDtypeStruct((_NSRC * _NY, _NX), jnp.float32),
    compiler_params=pltpu.CompilerParams(
        dimension_semantics=("arbitrary",),
    ),
)


def kernel(Y, X, y, x, dt=1.0):
    upd = jnp.asarray(dt, jnp.float32) * X.astype(jnp.float32).reshape(())
    upd16 = jnp.broadcast_to(upd, (_L,))
    yref = jax.new_ref(_tc_copy(Y.reshape(_NSRC * _NY, _NX)))
    _scatter_add(yref, y.astype(jnp.int32), x.astype(jnp.int32), upd16)
    return jax.freeze(yref).reshape(_NSRC, _NY, _NX)


# speedup vs baseline: 1.0163x; 1.0163x over previous
import jax
import jax.numpy as jnp
from jax import lax
from jax.experimental import pallas as pl
from jax.experimental.pallas import tpu as pltpu
from jax.experimental.pallas import tpu_sc as plsc

_NSRC = 32
_NY = 1024
_NX = 2048
_L = 16


def _sc_body(yref, pk_hbm, pk, tile, sem):
    cid = lax.axis_index("c")
    sid = lax.axis_index("s")
    wid = sid * 2 + cid

    pltpu.sync_copy(pk_hbm, pk)

    lanes = lax.iota(jnp.int32, _L)
    zero = jnp.zeros((_L,), jnp.int32)
    yi = jnp.int32(0)
    xi = jnp.int32(0)
    for c in range(_NSRC // _L):
        m = (lanes + c * _L) == wid
        yi = yi + jnp.sum(jnp.where(m, pk[pl.ds(c * _L, _L)], zero))
        xi = xi + jnp.sum(jnp.where(m, pk[pl.ds(_NSRC + c * _L, _L)], zero))
    upd = plsc.bitcast(pk[pl.ds(2 * _NSRC, _L)], jnp.float32)

    row0 = wid * _NY + (yi >> 3) * 8
    col0 = (xi >> 7) * 128
    ry = yi & 7
    c0 = (xi & 127) & ~15
    lane = xi & 15

    pltpu.async_copy(yref.at[pl.ds(row0, 8), pl.ds(col0, 128)], tile, sem).wait()
    sel = lanes == lane
    delta = jnp.where(sel, upd, jnp.float32(0.0))
    tile[ry, pl.ds(c0, _L)] = tile[ry, pl.ds(c0, _L)] + delta
    pltpu.async_copy(tile, yref.at[pl.ds(row0, 8), pl.ds(col0, 128)], sem).wait()


_scatter_add = pl.kernel(
    _sc_body,
    out_type=(),
    mesh=plsc.VectorSubcoreMesh(core_axis_name="c", subcore_axis_name="s"),
    scratch_types=[
        pltpu.VMEM((2 * _NSRC + _L,), jnp.int32),
        pltpu.VMEM((8, 128), jnp.float32),
        pltpu.SemaphoreType.DMA,
    ],
    compiler_params=pltpu.CompilerParams(
        use_tc_tiling_on_sc=True, needs_layout_passes=False
    ),
)


def kernel(Y, X, y, x, dt=1.0):
    upd = jnp.asarray(dt, jnp.float32) * X.astype(jnp.float32).reshape(())
    upd16 = jnp.broadcast_to(upd, (_L,))
    packed = jnp.concatenate(
        [
            y.astype(jnp.int32),
            x.astype(jnp.int32),
            lax.bitcast_convert_type(upd16, jnp.int32),
        ]
    )
    yref = jax.new_ref(Y.reshape(_NSRC * _NY, _NX))
    _scatter_add(yref, packed)
    return jax.freeze(yref).reshape(_NSRC, _NY, _NX)
